# baseline (speedup 1.0000x reference)
import jax
import jax.numpy as jnp
from jax import lax
from jax.experimental import pallas as pl
from jax.experimental.pallas import tpu as pltpu, tpu_sc as plsc

EMBED_DIM = 64
NC, NS = 2, 16
NW = NC * NS
CHUNK = 128


def _gather_body(table_hbm, ids_hbm, out_hbm, idx_v, rows_v, gsem):
    wid = lax.axis_index("s") * NC + lax.axis_index("c")
    n = ids_hbm.shape[0]
    b_per_w = n // NW
    base = wid * b_per_w
    pltpu.sync_copy(ids_hbm.at[pl.ds(base, b_per_w)], idx_v)
    n_chunks = b_per_w // CHUNK

    @pl.loop(0, n_chunks)
    def _step(j):
        off = j * CHUNK
        pltpu.async_copy(
            table_hbm.at[idx_v.at[pl.ds(off, CHUNK)]], rows_v, gsem
        ).wait()
        pltpu.sync_copy(rows_v, out_hbm.at[pl.ds(base + off, CHUNK)])


@jax.jit
def kernel(input_ids, table):
    b, s = input_ids.shape
    ids = input_ids.reshape(-1).astype(jnp.int32)
    n = ids.shape[0]
    b_per_w = n // NW
    call = pl.kernel(
        _gather_body,
        out_type=jax.ShapeDtypeStruct((n, EMBED_DIM), jnp.float32),
        mesh=plsc.VectorSubcoreMesh(
            core_axis_name="c", subcore_axis_name="s",
            num_cores=NC, num_subcores=NS,
        ),
        scratch_types=[
            pltpu.VMEM((b_per_w,), jnp.int32),
            pltpu.VMEM((CHUNK, EMBED_DIM), jnp.float32),
            pltpu.SemaphoreType.DMA,
        ],
        compiler_params=pltpu.CompilerParams(use_tc_tiling_on_sc=False),
    )
    out = call(table, ids)
    return out.reshape(b, s, EMBED_DIM)
